# baseline (speedup 1.0000x reference)
import jax
import jax.numpy as jnp
from jax.experimental import pallas as pl
from jax.experimental.pallas import tpu as pltpu

B, L, CLEN, E, TOP_K, D_MODEL = 256, 8, 128, 64, 2, 128
DIN = 3 * CLEN
R = B * L
EPS = 1e-9

_DN_T = (((0,), (0,)), ((), ()))


def _routing(logits, masks):
    maskf = (masks == 1).astype(jnp.float32)
    rowmax = jnp.max(logits, axis=1, keepdims=True)
    ex = jnp.exp(logits - rowmax)
    g = ex / jnp.sum(ex, axis=1, keepdims=True) * maskf

    iota = jax.lax.broadcasted_iota(jnp.int32, (B, E), 1)
    v1 = jnp.max(g, axis=1, keepdims=True)
    idx1 = jnp.min(jnp.where(g == v1, iota, E), axis=1, keepdims=True)
    oh1 = iota == idx1
    g2 = jnp.where(oh1, -1.0, g)
    v2 = jnp.max(g2, axis=1, keepdims=True)
    idx2 = jnp.min(jnp.where(g2 == v2, iota, E), axis=1, keepdims=True)
    oh2 = iota == idx2
    denom = v1 + v2 + EPS
    return (jnp.where(oh1, v1, 0.0) + jnp.where(oh2, v2, 0.0)) / denom


def _moe_kernel(logits_ref, masks_ref, flat_ref, W_ref, b_ref, out_ref):
    c = _routing(logits_ref[...], masks_ref[...])
    lane_b = jax.lax.broadcasted_iota(jnp.int32, (B, R), 1) // L
    sub_b = jax.lax.broadcasted_iota(jnp.int32, (B, R), 0)
    exp_mat = (lane_b == sub_b).astype(jnp.bfloat16)
    c2T = jax.lax.dot_general(
        c.astype(jnp.bfloat16), exp_mat, _DN_T,
        preferred_element_type=jnp.float32)

    xT = flat_ref[...].T.astype(jnp.bfloat16)
    acc = jax.lax.dot_general(
        b_ref[...], c2T, _DN_T, preferred_element_type=jnp.float32)
    for e in range(E):
        w = W_ref[e].astype(jnp.bfloat16)
        y = jax.lax.dot_general(w, xT, _DN_T,
                                preferred_element_type=jnp.float32)
        acc = acc + c2T[e:e + 1, :] * y
    out_ref[...] = acc.astype(jnp.bfloat16).T


def kernel(cycle_curve_data, logits, moe_masks, W, b):
    flat2 = cycle_curve_data.reshape(R, DIN)
    out = pl.pallas_call(
        _moe_kernel,
        out_shape=jax.ShapeDtypeStruct((R, D_MODEL), jnp.bfloat16),
    )(logits, moe_masks, flat2, W, b)
    return out.reshape(B, L, D_MODEL)
